# baseline (speedup 1.0000x reference)
import functools

import jax
import jax.numpy as jnp
import numpy as np
from jax import lax
from jax.experimental import pallas as pl
from jax.experimental.pallas import tpu as pltpu
from jax.experimental.pallas import tpu_sc as plsc

_EPS = 0.1
_G = 56


def _to_z_sc(n, p, hw, x_hbm, o_hbm, bufs, xbuf, sems):
    info = plsc.get_sparse_core_info()
    nc, ns = info.num_cores, info.num_subcores
    nw = nc * ns
    nch = p // _G
    per_w = n // nw

    wid = lax.axis_index("s") * nc + lax.axis_index("c")
    lanes = jnp.arange(16, dtype=jnp.int32)
    zeros16 = jnp.zeros((16,), jnp.float32)
    eps16 = jnp.full((16,), _EPS, jnp.float32)
    zero_i = jnp.zeros((16,), jnp.int32)
    g2 = hw - 16

    def _zero_slot(g, _):
        def _zr(h, _):
            for b in range(2):
                bufs[b, g, 0, h, pl.ds(0, 16)] = zeros16
                bufs[b, g, 0, h, pl.ds(g2, 16)] = zeros16
            return 0

        return lax.fori_loop(0, hw, _zr, 0)

    lax.fori_loop(0, _G, _zero_slot, 0)

    def _chunk_dmas(b, c, start):
        for s in range(per_w):
            batch = wid * per_w + s
            cp = pltpu.make_async_copy(
                bufs.at[b],
                o_hbm.at[batch, pl.ds(1 + c * _G, _G)],
                sems.at[b],
            )
            if start:
                cp.start()
            else:
                cp.wait()

    def _edit(b, c):
        for q in range(-(-_G // 16)):
            g = lanes + q * 16
            ok = g < _G
            e = c * _G + g
            e_old = e - 2 * _G
            plsc.store_scatter(
                bufs.at[b],
                [g, zero_i, e_old // hw, e_old % hw],
                zeros16,
                mask=ok & (e_old >= 0),
            )
            plsc.store_scatter(
                bufs.at[b], [g, zero_i, e // hw, e % hw], eps16, mask=ok
            )

    def _pair(t, _):
        for b in range(2):
            c = 2 * t + b

            @pl.when(c >= 2)
            def _wait_prev():
                _chunk_dmas(b, c - 2, start=False)

            _edit(b, c)
            _chunk_dmas(b, c, start=True)
        return 0

    lax.fori_loop(0, nch // 2, _pair, 0)

    for s in range(per_w):
        batch = wid * per_w + s
        pltpu.sync_copy(x_hbm.at[batch], xbuf)
        pltpu.sync_copy(xbuf, o_hbm.at[batch, 0])

    _chunk_dmas(0, nch - 2, start=False)
    _chunk_dmas(1, nch - 1, start=False)


def kernel(x):
    n = x.shape[0]
    inner = x.shape[1:]
    p = int(np.prod(inner))
    hw = inner[-1]
    mesh = plsc.VectorSubcoreMesh(core_axis_name="c", subcore_axis_name="s")
    out = pl.kernel(
        functools.partial(_to_z_sc, n, p, hw),
        out_type=jax.ShapeDtypeStruct((n, 1 + p) + tuple(inner), x.dtype),
        mesh=mesh,
        scratch_types=[
            pltpu.VMEM((2, _G) + tuple(inner), jnp.float32),
            pltpu.VMEM(tuple(inner), jnp.float32),
            pltpu.SemaphoreType.DMA((2,)),
        ],
        compiler_params=pltpu.CompilerParams(
            use_tc_tiling_on_sc=False, needs_layout_passes=False
        ),
    )(x)
    return out


# speedup vs baseline: 3.0869x; 3.0869x over previous
import jax
import jax.numpy as jnp
import numpy as np
from jax.experimental import pallas as pl
from jax.experimental.pallas import tpu as pltpu

_EPS = 0.1
_NSLOT = 4
_NSPLIT = 4


def _splits(tr):
    per = (tr // _NSPLIT) // 8 * 8
    starts = [8 + i * per for i in range(_NSPLIT)]
    sizes = [per] * (_NSPLIT - 1) + [tr - per * (_NSPLIT - 1)]
    return starts, sizes


def _fill_kernel(x_ref, o_hbm, tmpl, head, tsems, hsems):
    i = pl.program_id(0)
    n = pl.num_programs(0)
    p = tmpl.shape[1]
    tr = tmpl.shape[0]
    starts, sizes = _splits(tr)

    @pl.when(i == 0)
    def _init():
        r = jax.lax.broadcasted_iota(jnp.int32, (tr, p), 0)
        c = jax.lax.broadcasted_iota(jnp.int32, (tr, p), 1)
        tmpl[...] = jnp.where(c == r + 7, _EPS, 0.0).astype(tmpl.dtype)
        hr = jax.lax.broadcasted_iota(jnp.int32, (8, p), 0)
        hc = jax.lax.broadcasted_iota(jnp.int32, (8, p), 1)
        hbase = jnp.where(hr == hc + 1, _EPS, 0.0).astype(head.dtype)
        for s in range(_NSLOT):
            head[s] = hbase

    slot = jax.lax.rem(i, _NSLOT)

    def _tmpl_dmas(it, sl, start):
        for q in range(_NSPLIT):
            cp = pltpu.make_async_copy(
                tmpl.at[pl.ds(starts[q] - 8, sizes[q]), :],
                o_hbm.at[it, pl.ds(starts[q], sizes[q]), :],
                tsems.at[sl, q],
            )
            if start:
                cp.start()
            else:
                cp.wait()

    def _head_dma(it, sl, start):
        cp = pltpu.make_async_copy(
            head.at[sl], o_hbm.at[it, pl.ds(0, 8), :], hsems.at[sl]
        )
        if start:
            cp.start()
        else:
            cp.wait()

    @pl.when(i >= _NSLOT)
    def _wait_prev():
        _tmpl_dmas(i - _NSLOT, slot, start=False)
        _head_dma(i - _NSLOT, slot, start=False)

    head[slot, pl.ds(0, 1), :] = x_ref[0]
    _head_dma(i, slot, start=True)
    _tmpl_dmas(i, slot, start=True)

    @pl.when(i == n - 1)
    def _drain():
        for j in range(_NSLOT):
            it = n - _NSLOT + j
            _tmpl_dmas(it, it % _NSLOT, start=False)
            _head_dma(it, it % _NSLOT, start=False)


def kernel(x):
    n = x.shape[0]
    inner = x.shape[1:]
    p = int(np.prod(inner))
    xf = x.reshape(n, 1, p)
    out = pl.pallas_call(
        _fill_kernel,
        grid=(n,),
        in_specs=[pl.BlockSpec((1, 1, p), lambda i: (i, 0, 0))],
        out_specs=pl.BlockSpec(memory_space=pl.ANY),
        out_shape=jax.ShapeDtypeStruct((n, 1 + p, p), x.dtype),
        scratch_shapes=[
            pltpu.VMEM((p - 7, p), x.dtype),
            pltpu.VMEM((_NSLOT, 8, p), x.dtype),
            pltpu.SemaphoreType.DMA((_NSLOT, _NSPLIT)),
            pltpu.SemaphoreType.DMA((_NSLOT,)),
        ],
        compiler_params=pltpu.CompilerParams(
            dimension_semantics=("arbitrary",),
        ),
    )(xf)
    return out.reshape((n, 1 + p) + tuple(inner))
